# baseline (speedup 1.0000x reference)
import jax
import jax.numpy as jnp
from jax import lax
from jax.experimental import pallas as pl
from jax.experimental.pallas import tpu as pltpu
from jax.experimental.pallas import tpu_sc as plsc

_BATCH = 16384
_NFIELD = 26
_EMBED = 16
_FLAT = _BATCH * _NFIELD
_NCORES = 2
_NSUB = 16
_NW = _NCORES * _NSUB
_PER_W = _FLAT // _NW
_CHUNK = 128
_NCHUNK = _PER_W // _CHUNK
_LANES = 16


def _sc_lookup(x_hbm, tab_hbm, out_hbm, idx_v, rows_v, sem):
    wid = lax.axis_index("s") * _NCORES + lax.axis_index("c")
    base = wid * _PER_W

    @pl.loop(0, _NCHUNK)
    def _chunk(ci):
        cbase = base + ci * _CHUNK
        pltpu.sync_copy(x_hbm.at[pl.ds(cbase, _CHUNK)], idx_v)
        for j in range(_CHUNK // _LANES):
            pos = cbase + j * _LANES + lax.iota(jnp.int32, _LANES)
            field = lax.rem(pos, _NFIELD)
            idx_v[pl.ds(j * _LANES, _LANES)] = (
                idx_v[pl.ds(j * _LANES, _LANES)] + field * 100000
            )
        pltpu.async_copy(tab_hbm.at[idx_v], rows_v, sem).wait()
        pltpu.sync_copy(rows_v, out_hbm.at[pl.ds(cbase, _CHUNK)])


def kernel(x, table):
    x_flat = x.reshape(_FLAT).astype(jnp.int32)
    mesh = plsc.VectorSubcoreMesh(core_axis_name="c", subcore_axis_name="s")
    run = pl.kernel(
        _sc_lookup,
        out_type=jax.ShapeDtypeStruct((_FLAT, _EMBED), jnp.float32),
        mesh=mesh,
        scratch_types=[
            pltpu.VMEM((_CHUNK,), jnp.int32),
            pltpu.VMEM((_CHUNK, _EMBED), jnp.float32),
            pltpu.SemaphoreType.DMA,
        ],
        compiler_params=pltpu.CompilerParams(use_tc_tiling_on_sc=False),
    )
    out = run(x_flat, table)
    return out.reshape(_BATCH, _NFIELD, _EMBED)


# speedup vs baseline: 1.0615x; 1.0615x over previous
import jax
import jax.numpy as jnp
from jax import lax
from jax.experimental import pallas as pl
from jax.experimental.pallas import tpu as pltpu
from jax.experimental.pallas import tpu_sc as plsc

_BATCH = 16384
_NFIELD = 26
_EMBED = 16
_FLAT = _BATCH * _NFIELD
_NCORES = 2
_NSUB = 16
_NW = _NCORES * _NSUB
_PER_W = _FLAT // _NW
_CHUNK = 1024
_NCHUNK = _PER_W // _CHUNK
_LANES = 16


def _sc_lookup(x_hbm, tab_hbm, out_hbm, idx_v, rows_v, sem):
    wid = lax.axis_index("s") * _NCORES + lax.axis_index("c")
    base = wid * _PER_W

    @pl.loop(0, _NCHUNK)
    def _chunk(ci):
        cbase = base + ci * _CHUNK
        pltpu.sync_copy(x_hbm.at[pl.ds(cbase, _CHUNK)], idx_v)
        for j in range(_CHUNK // _LANES):
            pos = cbase + j * _LANES + lax.iota(jnp.int32, _LANES)
            field = lax.rem(pos, _NFIELD)
            idx_v[pl.ds(j * _LANES, _LANES)] = (
                idx_v[pl.ds(j * _LANES, _LANES)] + field * 100000
            )
        pltpu.async_copy(tab_hbm.at[idx_v], rows_v, sem).wait()
        pltpu.sync_copy(rows_v, out_hbm.at[pl.ds(cbase, _CHUNK)])


def kernel(x, table):
    x_flat = x.reshape(_FLAT).astype(jnp.int32)
    mesh = plsc.VectorSubcoreMesh(core_axis_name="c", subcore_axis_name="s")
    run = pl.kernel(
        _sc_lookup,
        out_type=jax.ShapeDtypeStruct((_FLAT, _EMBED), jnp.float32),
        mesh=mesh,
        scratch_types=[
            pltpu.VMEM((_CHUNK,), jnp.int32),
            pltpu.VMEM((_CHUNK, _EMBED), jnp.float32),
            pltpu.SemaphoreType.DMA,
        ],
        compiler_params=pltpu.CompilerParams(use_tc_tiling_on_sc=False),
    )
    out = run(x_flat, table)
    return out.reshape(_BATCH, _NFIELD, _EMBED)


# speedup vs baseline: 1.1907x; 1.1218x over previous
import jax
import jax.numpy as jnp
import numpy as np
from jax import lax
from jax.experimental import pallas as pl
from jax.experimental.pallas import tpu as pltpu
from jax.experimental.pallas import tpu_sc as plsc

_BATCH = 16384
_NFIELD = 26
_EMBED = 16
_NCORES = 2
_NSUB = 16
_NW = _NCORES * _NSUB
_ROWS_PER_W = _BATCH // _NW
_CB = 16
_CL = _CB * _NFIELD
_NSTEP = _CL // 16
_CPW = _ROWS_PER_W // _CB


def _sc_lookup(x_hbm, tab_hbm, out_hbm, xv, gidx, rows_v, out3, sem):
    wid = lax.axis_index("s") * _NCORES + lax.axis_index("c")
    row0 = wid * _ROWS_PER_W

    @pl.loop(0, _CPW)
    def _chunk(ci):
        b0 = row0 + ci * _CB
        pltpu.sync_copy(x_hbm.at[pl.ds(b0, _CB)], xv)
        for s in range(_NSTEP):
            k = s * 16 + lax.iota(jnp.int32, 16)
            b = lax.shift_right_logical(k * 40330, 20)
            f = k - b * _NFIELD
            gidx[pl.ds(s * 16, 16)] = f * 100000
        for s in range(_NSTEP):
            k = s * 16 + lax.iota(jnp.int32, 16)
            b = lax.shift_right_logical(k * 40330, 20)
            f = k - b * _NFIELD
            xval = plsc.load_gather(xv, [b, f])
            gidx[pl.ds(s * 16, 16)] = gidx[pl.ds(s * 16, 16)] + xval
        pltpu.async_copy(tab_hbm.at[gidx], rows_v, sem).wait()
        for bi in range(_CB):
            for f in range(_NFIELD):
                out3[bi, f, :] = rows_v[bi * _NFIELD + f, :]
        pltpu.sync_copy(out3, out_hbm.at[pl.ds(b0, _CB)])


def kernel(x, table):
    mesh = plsc.VectorSubcoreMesh(core_axis_name="c", subcore_axis_name="s")
    run = pl.kernel(
        _sc_lookup,
        out_type=jax.ShapeDtypeStruct((_BATCH, _NFIELD, _EMBED), jnp.float32),
        mesh=mesh,
        scratch_types=[
            pltpu.VMEM((_CB, _NFIELD), jnp.int32),
            pltpu.VMEM((_CL,), jnp.int32),
            pltpu.VMEM((_CL, _EMBED), jnp.float32),
            pltpu.VMEM((_CB, _NFIELD, _EMBED), jnp.float32),
            pltpu.SemaphoreType.DMA,
        ],
        compiler_params=pltpu.CompilerParams(
            use_tc_tiling_on_sc=False, needs_layout_passes=False
        ),
    )
    return run(x, table)


# speedup vs baseline: 1.2288x; 1.0320x over previous
import jax
import jax.numpy as jnp
import numpy as np
from jax import lax
from jax.experimental import pallas as pl
from jax.experimental.pallas import tpu as pltpu
from jax.experimental.pallas import tpu_sc as plsc

_BATCH = 16384
_NFIELD = 26
_EMBED = 16
_NCORES = 2
_NSUB = 16
_NW = _NCORES * _NSUB
_BPW = _BATCH // _NW
_LPW = _BPW * _NFIELD
_FLAT = _BATCH * _NFIELD
_GCHUNK = 1024


def _wid():
    return lax.axis_index("s") * _NCORES + lax.axis_index("c")


def _k1_index(xt_hbm, idx_hbm, xv, idxb):
    b0 = _wid() * _BPW
    for f in range(_NFIELD):
        pltpu.sync_copy(xt_hbm.at[f, pl.ds(b0, _BPW)], xv)
        for j in range(_BPW // 16):
            bloc = j * 16 + lax.iota(jnp.int32, 16)
            pos = bloc * _NFIELD + f
            val = xv[pl.ds(j * 16, 16)] + f * 100000
            plsc.store_scatter(idxb, [pos], val)
    pltpu.sync_copy(idxb, idx_hbm.at[pl.ds(_wid() * _LPW, _LPW)])


def _k2_gather(idx_hbm, tab_hbm, rows_hbm, idxv, rv, rv1, sem):
    base = _wid() * _LPW

    @pl.loop(0, _LPW // _GCHUNK)
    def _chunk(ci):
        cbase = base + ci * _GCHUNK
        pltpu.sync_copy(idx_hbm.at[pl.ds(cbase, _GCHUNK)], idxv)
        pltpu.async_copy(tab_hbm.at[idxv], rv, sem).wait()

        @pl.loop(0, _GCHUNK, unroll=8)
        def _row(r):
            rv1[pl.ds(r * _EMBED, _EMBED)] = rv[r]

        pltpu.sync_copy(
            rv1, rows_hbm.at[pl.ds(cbase * _EMBED, _GCHUNK * _EMBED)]
        )


_CB3 = 128
_FE = _NFIELD * _EMBED


def _k3_pack(rows_hbm, out_hbm, rbuf, o2):
    b0 = _wid() * _BPW

    @pl.loop(0, _BPW // _CB3)
    def _chunk(ci):
        bb = b0 + ci * _CB3
        pltpu.sync_copy(
            rows_hbm.at[pl.ds(bb * _FE, _CB3 * _FE)], rbuf
        )

        @pl.loop(0, _CB3 // 16)
        def _jstep(j):
            base = j * 16 * _FE + lax.iota(jnp.int32, 16) * _FE
            for fe in range(_FE):
                vals = plsc.load_gather(rbuf, [base + fe])
                o2[fe, pl.ds(j * 16, 16)] = vals

        pltpu.sync_copy(o2, out_hbm.at[:, pl.ds(bb, _CB3)])


def kernel(x, table):
    mesh = plsc.VectorSubcoreMesh(core_axis_name="c", subcore_axis_name="s")
    cp_tc = pltpu.CompilerParams(needs_layout_passes=False)
    cp_dense = pltpu.CompilerParams(
        use_tc_tiling_on_sc=False, needs_layout_passes=False
    )

    k1 = pl.kernel(
        _k1_index,
        out_type=jax.ShapeDtypeStruct((_FLAT,), jnp.int32),
        mesh=mesh,
        scratch_types=[
            pltpu.VMEM((_BPW,), jnp.int32),
            pltpu.VMEM((_LPW,), jnp.int32),
        ],
        compiler_params=cp_tc,
    )
    k2 = pl.kernel(
        _k2_gather,
        out_type=jax.ShapeDtypeStruct((_FLAT * _EMBED,), jnp.float32),
        mesh=mesh,
        scratch_types=[
            pltpu.VMEM((_GCHUNK,), jnp.int32),
            pltpu.VMEM((_GCHUNK, _EMBED), jnp.float32),
            pltpu.VMEM((_GCHUNK * _EMBED,), jnp.float32),
            pltpu.SemaphoreType.DMA,
        ],
        compiler_params=cp_dense,
    )
    k3 = pl.kernel(
        _k3_pack,
        out_type=jax.ShapeDtypeStruct((_FE, _BATCH), jnp.float32),
        mesh=mesh,
        scratch_types=[
            pltpu.VMEM((_CB3 * _FE,), jnp.float32),
            pltpu.VMEM((_FE, _CB3), jnp.float32),
        ],
        compiler_params=cp_tc,
    )

    idx = k1(x.T)
    rows = k2(idx, table)
    outp = k3(rows)
    return outp.reshape(_NFIELD, _EMBED, _BATCH).transpose(2, 0, 1)


# speedup vs baseline: 1.2298x; 1.0009x over previous
import jax
import jax.numpy as jnp
import numpy as np
from jax import lax
from jax.experimental import pallas as pl
from jax.experimental.pallas import tpu as pltpu
from jax.experimental.pallas import tpu_sc as plsc

_BATCH = 16384
_NFIELD = 26
_EMBED = 16
_NCORES = 2
_NSUB = 16
_NW = _NCORES * _NSUB
_BPW = _BATCH // _NW
_LPW = _BPW * _NFIELD
_FLAT = _BATCH * _NFIELD
_GCHUNK = 1024


def _wid():
    return lax.axis_index("s") * _NCORES + lax.axis_index("c")


def _k1_index(xt_hbm, idx_hbm, xv, idxb):
    b0 = _wid() * _BPW
    for f in range(_NFIELD):
        pltpu.sync_copy(xt_hbm.at[f, pl.ds(b0, _BPW)], xv)
        for j in range(_BPW // 16):
            bloc = j * 16 + lax.iota(jnp.int32, 16)
            pos = bloc * _NFIELD + f
            val = xv[pl.ds(j * 16, 16)] + f * 100000
            plsc.store_scatter(idxb, [pos], val)
    pltpu.sync_copy(idxb, idx_hbm.at[pl.ds(_wid() * _LPW, _LPW)])


def _k2_gather(idx_hbm, tab_hbm, rows_hbm, idxv, rv, rv1, sem):
    base = _wid() * _LPW

    @pl.loop(0, _LPW // _GCHUNK)
    def _chunk(ci):
        cbase = base + ci * _GCHUNK
        pltpu.sync_copy(idx_hbm.at[pl.ds(cbase, _GCHUNK)], idxv)
        pltpu.async_copy(tab_hbm.at[idxv], rv, sem).wait()

        @pl.loop(0, _GCHUNK, unroll=8)
        def _row(r):
            rv1[pl.ds(r * _EMBED, _EMBED)] = rv[r]

        pltpu.sync_copy(
            rv1, rows_hbm.at[pl.ds(cbase * _EMBED, _GCHUNK * _EMBED)]
        )


_CB3 = 128
_FE = _NFIELD * _EMBED


def _k3_pack(rows_hbm, out_hbm, rbuf, o2):
    b0 = _wid() * _BPW

    @pl.loop(0, _BPW // _CB3)
    def _chunk(ci):
        bb = b0 + ci * _CB3
        pltpu.sync_copy(
            rows_hbm.at[pl.ds(bb * _FE, _CB3 * _FE)], rbuf
        )

        @pl.loop(0, _CB3 // 16)
        def _jstep(j):
            base = j * 16 * _FE + lax.iota(jnp.int32, 16) * _FE
            for fe in range(_FE):
                vals = plsc.load_gather(rbuf, [base + fe])
                o2[fe, pl.ds(j * 16, 16)] = vals

        pltpu.sync_copy(o2, out_hbm.at[:, pl.ds(bb, _CB3)])


def kernel(x, table):
    mesh = plsc.VectorSubcoreMesh(core_axis_name="c", subcore_axis_name="s")
    cp_tc = pltpu.CompilerParams(
        use_tc_tiling_on_sc=True, needs_layout_passes=False
    )
    cp_dense = pltpu.CompilerParams(
        use_tc_tiling_on_sc=False, needs_layout_passes=False
    )

    k1 = pl.kernel(
        _k1_index,
        out_type=jax.ShapeDtypeStruct((_FLAT,), jnp.int32),
        mesh=mesh,
        scratch_types=[
            pltpu.VMEM((_BPW,), jnp.int32),
            pltpu.VMEM((_LPW,), jnp.int32),
        ],
        compiler_params=cp_tc,
    )
    k2 = pl.kernel(
        _k2_gather,
        out_type=jax.ShapeDtypeStruct((_FLAT * _EMBED,), jnp.float32),
        mesh=mesh,
        scratch_types=[
            pltpu.VMEM((_GCHUNK,), jnp.int32),
            pltpu.VMEM((_GCHUNK, _EMBED), jnp.float32),
            pltpu.VMEM((_GCHUNK * _EMBED,), jnp.float32),
            pltpu.SemaphoreType.DMA,
        ],
        compiler_params=cp_dense,
    )
    k3 = pl.kernel(
        _k3_pack,
        out_type=jax.ShapeDtypeStruct((_FE, _BATCH), jnp.float32),
        mesh=mesh,
        scratch_types=[
            pltpu.VMEM((_CB3 * _FE,), jnp.float32),
            pltpu.VMEM((_FE, _CB3), jnp.float32),
        ],
        compiler_params=cp_tc,
    )

    idx = k1(x.T)
    rows = k2(idx, table)
    outp = k3(rows)
    return outp.reshape(_NFIELD, _EMBED, _BATCH).transpose(2, 0, 1)
